# baseline (speedup 1.0000x reference)
import jax
import jax.numpy as jnp
from jax import lax
from jax.experimental import pallas as pl
from jax.experimental.pallas import tpu as pltpu
from jax.experimental.pallas import tpu_sc as plsc

BATCH = 16384
TITLE_DIM = 32
LOC_DIM = 31
OUT_DIM = TITLE_DIM + 1 + LOC_DIM

_NC = 2
_NS = 16
_NW = _NC * _NS
_BW = BATCH // _NW
_R = 16


def _body(title_hbm, grade_hbm, loc_hbm, ttab_hbm, ltab_hbm, out_hbm,
          tidx_v, lidx_v, g_v, trows_v, stage_v, cflat_v, sem_t, sem_ring):
    wid = lax.axis_index("s") * _NC + lax.axis_index("c")
    base = wid * _BW

    pltpu.sync_copy(title_hbm.at[pl.ds(base, _BW)], tidx_v)
    pltpu.sync_copy(loc_hbm.at[pl.ds(base, _BW)], lidx_v.at[pl.ds(0, _BW)])
    pltpu.sync_copy(grade_hbm.at[pl.ds(base, _BW)], g_v.at[pl.ds(0, _BW)])

    ct = pltpu.async_copy(ttab_hbm.at[tidx_v], trows_v, sem_t)

    def loc_dma(r, slot):
        i = lidx_v[pl.ds(r, 16)][0]
        pltpu.make_async_copy(ltab_hbm.at[i], stage_v.at[slot],
                              sem_ring.at[slot]).start()

    for j in range(_R):
        loc_dma(j, j)

    ct.wait()

    lane = lax.iota(jnp.int32, 16)

    def group(gidx, _):
        r0 = gidx * _R
        for j in range(_R):
            r = r0 + j
            pltpu.make_async_copy(ltab_hbm.at[0], stage_v.at[j],
                                  sem_ring.at[j]).wait()
            o = r * OUT_DIM
            cflat_v[pl.ds(o, 16)] = trows_v[r, pl.ds(0, 16)]
            cflat_v[pl.ds(o + 16, 16)] = trows_v[r, pl.ds(16, 16)]
            cflat_v[pl.ds(o + 33, 16)] = stage_v[j, pl.ds(0, 16)]
            cflat_v[pl.ds(o + 48, 16)] = stage_v[j, pl.ds(15, 16)]
            g = g_v[pl.ds(r, 16)][0]
            m = cflat_v[pl.ds(o + 32, 16)]
            cflat_v[pl.ds(o + 32, 16)] = jnp.where(lane == 0, g, m)

            @pl.when(r + _R < _BW)
            def _():
                loc_dma(r + _R, j)
        return 0

    lax.fori_loop(0, _BW // _R, group, 0)

    pltpu.sync_copy(cflat_v, out_hbm.at[pl.ds(base * OUT_DIM, _BW * OUT_DIM)])


@jax.jit
def _run(title, grade, location, title_table, location_table):
    mesh = plsc.VectorSubcoreMesh(core_axis_name="c", subcore_axis_name="s")
    k = pl.kernel(
        _body,
        out_type=jax.ShapeDtypeStruct((BATCH * OUT_DIM,), jnp.float32),
        mesh=mesh,
        compiler_params=pltpu.CompilerParams(use_tc_tiling_on_sc=False),
        scratch_types=[
            pltpu.VMEM((_BW,), jnp.int32),
            pltpu.VMEM((_BW + 16,), jnp.int32),
            pltpu.VMEM((_BW + 16,), jnp.float32),
            pltpu.VMEM((_BW, TITLE_DIM), jnp.float32),
            pltpu.VMEM((_R, LOC_DIM), jnp.float32),
            pltpu.VMEM((_BW * OUT_DIM,), jnp.float32),
            pltpu.SemaphoreType.DMA,
            pltpu.SemaphoreType.DMA((_R,)),
        ],
    )
    flat = k(title, grade, location, title_table, location_table)
    return flat.reshape(BATCH, OUT_DIM)


def kernel(title, grade, location, title_table, location_table):
    return _run(title.astype(jnp.int32), grade.astype(jnp.float32),
                location.astype(jnp.int32), title_table, location_table)


# speedup vs baseline: 1.0790x; 1.0790x over previous
import jax
import jax.numpy as jnp
from jax import lax
from jax.experimental import pallas as pl
from jax.experimental.pallas import tpu as pltpu
from jax.experimental.pallas import tpu_sc as plsc

BATCH = 16384
TITLE_DIM = 32
LOC_DIM = 31
OUT_DIM = TITLE_DIM + 1 + LOC_DIM

_NC = 2
_NS = 16
_NW = _NC * _NS
_BW = BATCH // _NW


def _body(title_hbm, grade_hbm, loc_hbm, ttab_hbm, ltab_hbm, out_hbm,
          tidx_v, lidx_v, g_v, trows_v, lrows_v, c_v, sem_t, sem_l):
    wid = lax.axis_index("s") * _NC + lax.axis_index("c")
    base = wid * _BW

    pltpu.sync_copy(title_hbm.at[pl.ds(base, _BW)], tidx_v)
    pltpu.sync_copy(loc_hbm.at[pl.ds(base, _BW)], lidx_v)
    pltpu.sync_copy(grade_hbm.at[pl.ds(base, _BW)], g_v.at[pl.ds(0, _BW)])

    ct = pltpu.async_copy(ttab_hbm.at[tidx_v], trows_v, sem_t)
    cl = pltpu.async_copy(ltab_hbm.at[lidx_v], lrows_v, sem_l)
    ct.wait()
    cl.wait()

    lane = lax.iota(jnp.int32, 16)

    def row(r, _):
        c_v[r, pl.ds(0, 16)] = trows_v[r, pl.ds(0, 16)]
        c_v[r, pl.ds(16, 16)] = trows_v[r, pl.ds(16, 16)]
        c_v[r, pl.ds(33, 16)] = lrows_v[r, pl.ds(0, 16)]
        c_v[r, pl.ds(48, 16)] = lrows_v[r, pl.ds(15, 16)]
        g = g_v[pl.ds(r, 16)][0]
        m = c_v[r, pl.ds(32, 16)]
        c_v[r, pl.ds(32, 16)] = jnp.where(lane == 0, g, m)
        return 0

    lax.fori_loop(0, _BW, row, 0)

    pltpu.sync_copy(c_v, out_hbm.at[pl.ds(base, _BW)])


@jax.jit
def _run(title, grade, location, title_table, location_table):
    mesh = plsc.VectorSubcoreMesh(core_axis_name="c", subcore_axis_name="s")
    ltab32 = jnp.pad(location_table, ((0, 0), (0, 1)))
    k = pl.kernel(
        _body,
        out_type=jax.ShapeDtypeStruct((BATCH, OUT_DIM), jnp.float32),
        mesh=mesh,
        compiler_params=pltpu.CompilerParams(use_tc_tiling_on_sc=False),
        scratch_types=[
            pltpu.VMEM((_BW,), jnp.int32),
            pltpu.VMEM((_BW,), jnp.int32),
            pltpu.VMEM((_BW + 16,), jnp.float32),
            pltpu.VMEM((_BW, TITLE_DIM), jnp.float32),
            pltpu.VMEM((_BW, LOC_DIM + 1), jnp.float32),
            pltpu.VMEM((_BW, OUT_DIM), jnp.float32),
            pltpu.SemaphoreType.DMA,
            pltpu.SemaphoreType.DMA,
        ],
    )
    return k(title, grade, location, title_table, ltab32)


def kernel(title, grade, location, title_table, location_table):
    return _run(title.astype(jnp.int32), grade.astype(jnp.float32),
                location.astype(jnp.int32), title_table, location_table)


# speedup vs baseline: 1.1555x; 1.0708x over previous
import jax
import jax.numpy as jnp
from jax import lax
from jax.experimental import pallas as pl
from jax.experimental.pallas import tpu as pltpu
from jax.experimental.pallas import tpu_sc as plsc

BATCH = 16384
TITLE_DIM = 32
LOC_DIM = 31
OUT_DIM = TITLE_DIM + 1 + LOC_DIM

_NC = 2
_NS = 16
_NW = _NC * _NS
_BW = BATCH // _NW
_R = 8


def _body(title_hbm, grade_hbm, loc_hbm, ttab_hbm, ltab_hbm, out_hbm,
          tidx_v, lidx_v, g_v, c_v, *stages_and_sems):
    tstages = stages_and_sems[0:_R]
    lstages = stages_and_sems[_R:2 * _R]
    sem_t = stages_and_sems[2 * _R]
    sem_l = stages_and_sems[2 * _R + 1]

    wid = lax.axis_index("s") * _NC + lax.axis_index("c")
    base = wid * _BW

    pltpu.sync_copy(title_hbm.at[pl.ds(base, _BW)], tidx_v.at[pl.ds(0, _BW)])
    pltpu.sync_copy(loc_hbm.at[pl.ds(base, _BW)], lidx_v.at[pl.ds(0, _BW)])
    pltpu.sync_copy(grade_hbm.at[pl.ds(base, _BW)], g_v.at[pl.ds(0, _BW)])

    def fetch(r, slot):
        ti = tidx_v[pl.ds(r, 16)][0]
        li = lidx_v[pl.ds(r, 16)][0]
        pltpu.make_async_copy(ttab_hbm.at[ti], tstages[slot],
                              sem_t.at[slot]).start()
        pltpu.make_async_copy(ltab_hbm.at[li], lstages[slot],
                              sem_l.at[slot]).start()

    for j in range(_R):
        fetch(j, j)

    lane = lax.iota(jnp.int32, 16)

    def group(gidx, _):
        r0 = gidx * _R
        for j in range(_R):
            r = r0 + j
            pltpu.make_async_copy(ttab_hbm.at[0], tstages[j],
                                  sem_t.at[j]).wait()
            pltpu.make_async_copy(ltab_hbm.at[0], lstages[j],
                                  sem_l.at[j]).wait()
            o = r * OUT_DIM
            c_v[pl.ds(o, 16)] = tstages[j][pl.ds(0, 16)]
            c_v[pl.ds(o + 16, 16)] = tstages[j][pl.ds(16, 16)]
            c_v[pl.ds(o + 33, 16)] = lstages[j][pl.ds(0, 16)]
            c_v[pl.ds(o + 48, 16)] = lstages[j][pl.ds(15, 16)]
            g = g_v[pl.ds(r, 16)][0]
            m = c_v[pl.ds(o + 32, 16)]
            c_v[pl.ds(o + 32, 16)] = jnp.where(lane == 0, g, m)

            @pl.when(r + _R < _BW)
            def _():
                fetch(r + _R, j)
        return 0

    lax.fori_loop(0, _BW // _R, group, 0)

    pltpu.sync_copy(c_v, out_hbm.at[pl.ds(base * OUT_DIM, _BW * OUT_DIM)])


@jax.jit
def _run(title, grade, location, title_table, location_table):
    mesh = plsc.VectorSubcoreMesh(core_axis_name="c", subcore_axis_name="s")
    scratch = (
        [pltpu.VMEM((_BW + 16,), jnp.int32),
         pltpu.VMEM((_BW + 16,), jnp.int32),
         pltpu.VMEM((_BW + 16,), jnp.float32),
         pltpu.VMEM((_BW * OUT_DIM,), jnp.float32)]
        + [pltpu.VMEM((TITLE_DIM,), jnp.float32) for _ in range(_R)]
        + [pltpu.VMEM((LOC_DIM,), jnp.float32) for _ in range(_R)]
        + [pltpu.SemaphoreType.DMA((_R,)), pltpu.SemaphoreType.DMA((_R,))]
    )
    k = pl.kernel(
        _body,
        out_type=jax.ShapeDtypeStruct((BATCH * OUT_DIM,), jnp.float32),
        mesh=mesh,
        compiler_params=pltpu.CompilerParams(use_tc_tiling_on_sc=True),
        scratch_types=scratch,
    )
    flat = k(title, grade, location, title_table, location_table)
    return flat.reshape(BATCH, OUT_DIM)


def kernel(title, grade, location, title_table, location_table):
    return _run(title.astype(jnp.int32), grade.astype(jnp.float32),
                location.astype(jnp.int32), title_table, location_table)
